# baseline (speedup 1.0000x reference)
import functools

import jax
import jax.numpy as jnp
from jax import lax
from jax.experimental import pallas as pl
from jax.experimental.pallas import tpu as pltpu
from jax.experimental.pallas import tpu_sc as plsc

BATCH = 16384
EMB = 32
NUM_CORES = 2
NUM_SUBCORES = 16
NUM_WORKERS = NUM_CORES * NUM_SUBCORES
ROWS_PER_WORKER = BATCH // NUM_WORKERS

_sc_mesh = plsc.VectorSubcoreMesh(core_axis_name="c", subcore_axis_name="s")


@functools.partial(
    pl.kernel,
    mesh=_sc_mesh,
    compiler_params=pltpu.CompilerParams(use_tc_tiling_on_sc=False),
    out_type=jax.ShapeDtypeStruct((BATCH, EMB), jnp.float32),
    scratch_types=[
        pltpu.VMEM((ROWS_PER_WORKER,), jnp.int32),
        pltpu.VMEM((ROWS_PER_WORKER, EMB), jnp.float32),
        pltpu.SemaphoreType.DMA,
    ],
)
def _sc_gather(table_hbm, idx_hbm, out_hbm, idx_v, rows_v, sem):
    wid = lax.axis_index("s") * NUM_CORES + lax.axis_index("c")
    base = wid * ROWS_PER_WORKER
    pltpu.sync_copy(idx_hbm.at[pl.ds(base, ROWS_PER_WORKER)], idx_v)
    pltpu.async_copy(table_hbm.at[idx_v], rows_v, sem).wait()
    pltpu.sync_copy(rows_v, out_hbm.at[pl.ds(base, ROWS_PER_WORKER)])


def _dot_sigmoid_body(m_ref, u_ref, w_ref, b_ref, o_ref):
    s = jnp.sum(m_ref[...] * u_ref[...], axis=1, keepdims=True)
    o_ref[...] = jax.nn.sigmoid(s * w_ref[0, 0] + b_ref[0])


_TC_BLOCK = 2048


def _tc_dot_sigmoid(movie_emb, uemb, W, b):
    grid = BATCH // _TC_BLOCK
    return pl.pallas_call(
        _dot_sigmoid_body,
        grid=(grid,),
        in_specs=[
            pl.BlockSpec((_TC_BLOCK, EMB), lambda i: (i, 0)),
            pl.BlockSpec((_TC_BLOCK, EMB), lambda i: (i, 0)),
            pl.BlockSpec(memory_space=pltpu.SMEM),
            pl.BlockSpec(memory_space=pltpu.SMEM),
        ],
        out_specs=pl.BlockSpec((_TC_BLOCK, 1), lambda i: (i, 0)),
        out_shape=jax.ShapeDtypeStruct((BATCH, 1), jnp.float32),
    )(movie_emb, uemb, W, b)


@jax.jit
def kernel(user_ids, movie_emb, table, W, b):
    uemb = _sc_gather(table, user_ids.astype(jnp.int32))
    return _tc_dot_sigmoid(movie_emb, uemb, W, b)
